# baseline (speedup 1.0000x reference)
import functools

import jax
import jax.numpy as jnp
from jax import lax
from jax.experimental import pallas as pl
from jax.experimental.pallas import tpu as pltpu
from jax.experimental.pallas import tpu_sc as plsc

N_SPOT, N_GENE = 50000, 2000
E_EDGE = 800000
IN_CH, HID, HEADS, DH = 128, 64, 4, 16
OUT_CH = 16

HALF = N_SPOT // 2
TRASH = HALF
ACC_R = HALF + 8
ROW_W = 40
HW = 32
CH = 128
CT = 400
E_PAD = CT * 16 * CH
SUP = 16
NSUP = CT // SUP
SDW = SUP * 2 * CH
ROWB = 40
NBLK = HALF // ROWB


def _make_edge_kernel(sel_off):
    mesh = plsc.VectorSubcoreMesh(core_axis_name="c", subcore_axis_name="s")

    @functools.partial(
        pl.kernel,
        out_type=[jax.ShapeDtypeStruct((N_SPOT, HW), jnp.float32),
                  jax.ShapeDtypeStruct((N_SPOT, HW), jnp.float32)],
        mesh=mesh,
        scratch_types=[
            pltpu.VMEM((2 * SDW,), jnp.int32),
            pltpu.VMEM((2, CH), jnp.int32),
            pltpu.VMEM((2 * CH, HW), jnp.float32),
            pltpu.VMEM((2 * CH, 16), jnp.float32),
            pltpu.VMEM((2 * CH, 16), jnp.float32),
            pltpu.VMEM((2 * CH, ROW_W), jnp.float32),
            pltpu.VMEM((ROWB, ROW_W), jnp.float32),
            pltpu.VMEM((ROWB, HW), jnp.float32),
            pltpu.VMEM_SHARED((ACC_R, ROW_W), jnp.float32),
            pltpu.SemaphoreType.DMA,
            pltpu.SemaphoreType.DMA((2,)),
            pltpu.SemaphoreType.DMA((2,)),
            pltpu.SemaphoreType.DMA((2,)),
            pltpu.SemaphoreType.DMA((2,)),
        ],
        compiler_params=pltpu.CompilerParams(use_tc_tiling_on_sc=False),
    )
    def ek(h01_hbm, h23_hbm, asrc_hbm, adst_hbm, sd_hbm,
           out0_hbm, out1_hbm,
           sd_v, dstc_v, h_v, as_v, ad_v, msg_v, fin_v, fo_v,
           acc, sem_sd, sem_h, sem_a, sem_d, sem_sc):
        c = lax.axis_index("c")
        sid = lax.axis_index("s")
        lo = c * HALF
        tile_base = sid * CT * 2 * CH

        btrips = 39 + jnp.where(sid < NBLK - 39 * 16, 1, 0)
        sel_a = lax.iota(jnp.int32, 16) % 4 + sel_off
        sel_e = jnp.maximum(lax.iota(jnp.int32, 16) - 8, 0)
        zv = jnp.zeros((16,), jnp.float32)

        def sd_off(j):
            return ((j // SUP) % 2) * SDW + (j % SUP) * 2 * CH

        def gathers(j, par, h_hbm, wait):
            off = sd_off(j)
            src_idx = sd_v.at[pl.ds(off, CH)]
            dst_idx = sd_v.at[pl.ds(off + CH, CH)]
            for tab, idx, buf, sem in (
                    (h_hbm, src_idx, h_v, sem_h),
                    (asrc_hbm, src_idx, as_v, sem_a),
                    (adst_hbm, dst_idx, ad_v, sem_d)):
                if wait:
                    pltpu.make_async_copy(tab.at[idx],
                                          buf.at[pl.ds(par * CH, CH)],
                                          sem.at[par]).wait()
                else:
                    pltpu.async_copy(tab.at[idx],
                                     buf.at[pl.ds(par * CH, CH)],
                                     sem.at[par])

        def scatter_wait(par):
            pltpu.make_async_copy(msg_v.at[pl.ds(par * CH, CH)],
                                  acc.at[dstc_v.at[par]],
                                  sem_sc.at[par]).wait()

        for phase, (h_hbm, out_hbm) in enumerate(
                ((h01_hbm, out0_hbm), (h23_hbm, out1_hbm))):
            for r in range(ROWB):
                for g0 in (0, 16, 24):
                    fin_v[r, pl.ds(g0, 16)] = zv

            def zblk(j, carry):
                b = sid + 16 * j
                pltpu.sync_copy(fin_v, acc.at[pl.ds(b * ROWB, ROWB)])
                return carry

            lax.fori_loop(0, btrips, zblk, 0)
            plsc.subcore_barrier()

            pltpu.sync_copy(sd_hbm.at[pl.ds(tile_base, SDW)],
                            sd_v.at[pl.ds(0, SDW)])
            pltpu.async_copy(sd_hbm.at[pl.ds(tile_base + SDW, SDW)],
                             sd_v.at[pl.ds(SDW, SDW)], sem_sd)
            gathers(0, 0, h_hbm, wait=False)

            def chunk_body(j, carry):
                par = j % 2
                npar = 1 - par

                @pl.when(j >= 2)
                def _():
                    scatter_wait(par)

                @pl.when(j < CT - 1)
                def _():
                    @pl.when((j + 1) % SUP == 0)
                    def _():
                        sp = ((j + 1) // SUP) % 2
                        pltpu.make_async_copy(
                            sd_hbm.at[pl.ds(
                                tile_base + ((j + 1) // SUP) * SDW, SDW)],
                            sd_v.at[pl.ds(sp * SDW, SDW)], sem_sd).wait()
                    gathers(j + 1, npar, h_hbm, wait=False)

                gathers(j, par, h_hbm, wait=True)

                doff = sd_off(j) + CH
                for g in range(CH // 16):
                    d = sd_v[pl.ds(doff + g * 16, 16)] - lo
                    ok = (d >= 0) & (d < HALF)
                    dstc_v[par, pl.ds(g * 16, 16)] = jnp.where(ok, d, TRASH)

                def edge_body(e4, carry2):
                    for u in range(4):
                        e = par * CH + e4 * 4 + u
                        ad_al = ad_v[e, :].at[sel_a].get(
                            mode="promise_in_bounds")
                        a = as_v[e, :] + ad_al
                        a = jnp.maximum(a, 0.2 * a)
                        ex = jnp.exp(a)
                        evec = ex.at[sel_e].get(mode="promise_in_bounds")
                        msg_v[e, pl.ds(24, 16)] = evec
                        for hh in range(2):
                            bi = jnp.full((16,), 2 * phase + hh, jnp.int32)
                            w = ex.at[bi].get(mode="promise_in_bounds")
                            msg_v[e, pl.ds(DH * hh, 16)] = (
                                h_v[e, pl.ds(DH * hh, 16)] * w)
                    return carry2

                lax.fori_loop(0, CH // 4, edge_body, 0)
                pltpu.async_copy(msg_v.at[pl.ds(par * CH, CH)],
                                 acc.at[dstc_v.at[par]],
                                 sem_sc.at[par], add=True)

                @pl.when(((j + 1) % SUP == 0) & (j + SUP + 1 < CT))
                def _():
                    s1 = (j + 1) // SUP + 1
                    pltpu.async_copy(
                        sd_hbm.at[pl.ds(tile_base + s1 * SDW, SDW)],
                        sd_v.at[pl.ds((s1 % 2) * SDW, SDW)], sem_sd)

                return carry

            lax.fori_loop(0, CT, chunk_body, 0)
            scatter_wait(0)
            scatter_wait(1)
            plsc.subcore_barrier()

            def fin_body(j, carry):
                b = sid + 16 * j
                r0 = b * ROWB
                pltpu.sync_copy(acc.at[pl.ds(r0, ROWB)], fin_v)

                def row_body(r, carry2):
                    exr = fin_v[r, pl.ds(24, 16)]
                    for hh in range(2):
                        bi = jnp.full((16,), 8 + 2 * phase + hh, jnp.int32)
                        den = exr.at[bi].get(mode="promise_in_bounds") + 1e-16
                        o = fin_v[r, pl.ds(DH * hh, 16)] / den
                        fo_v[r, pl.ds(DH * hh, 16)] = jnp.maximum(o, 0.0)
                    return carry2

                lax.fori_loop(0, ROWB, row_body, 0)
                pltpu.sync_copy(fo_v, out_hbm.at[pl.ds(lo + r0, ROWB)])
                return carry

            lax.fori_loop(0, btrips, fin_body, 0)

    return ek


def _prep_nodes(x, W, b2, A, n_rows, blk):
    def body(x_ref, w_ref, b_ref, a_ref, h0_ref, h1_ref, al_ref):
        h = jnp.dot(x_ref[...], w_ref[...],
                    preferred_element_type=jnp.float32) + b_ref[...]
        h0_ref[...] = h[:, :HW]
        h1_ref[...] = h[:, HW:]
        al_ref[...] = jnp.dot(h, a_ref[...], preferred_element_type=jnp.float32)

    grid = n_rows // blk
    return pl.pallas_call(
        body,
        grid=(grid,),
        in_specs=[
            pl.BlockSpec((blk, IN_CH), lambda i: (i, 0)),
            pl.BlockSpec((IN_CH, HID), lambda i: (0, 0)),
            pl.BlockSpec((1, HID), lambda i: (0, 0)),
            pl.BlockSpec((HID, 16), lambda i: (0, 0)),
        ],
        out_specs=[
            pl.BlockSpec((blk, HW), lambda i: (i, 0)),
            pl.BlockSpec((blk, HW), lambda i: (i, 0)),
            pl.BlockSpec((blk, 16), lambda i: (i, 0)),
        ],
        out_shape=[
            jax.ShapeDtypeStruct((n_rows, HW), jnp.float32),
            jax.ShapeDtypeStruct((n_rows, HW), jnp.float32),
            jax.ShapeDtypeStruct((n_rows, 16), jnp.float32),
        ],
    )(x, W, b2, A)


def _semantic_score(ss0, ss1, gs0, gs1, kW, kb2, q2, blk):
    def body(s0_ref, s1_ref, g0_ref, g1_ref, kw_ref, kb_ref, q_ref, o_ref):
        i = pl.program_id(0)

        @pl.when(i == 0)
        def _():
            o_ref[...] = jnp.zeros_like(o_ref)

        ss = jnp.concatenate([s0_ref[...], s1_ref[...]], axis=1)
        gs = jnp.concatenate([g0_ref[...], g1_ref[...]], axis=1)
        t0 = jnp.tanh(jnp.dot(ss, kw_ref[...],
                              preferred_element_type=jnp.float32) + kb_ref[...])
        t1 = jnp.tanh(jnp.dot(gs, kw_ref[...],
                              preferred_element_type=jnp.float32) + kb_ref[...])
        sc0 = jnp.sum(t0 * q_ref[...])
        sc1 = jnp.sum(t1 * q_ref[...])
        rows = lax.broadcasted_iota(jnp.int32, (8, 128), 0)
        upd = jnp.where(rows == 0, sc0, 0.0) + jnp.where(rows == 1, sc1, 0.0)
        o_ref[...] = o_ref[...] + upd

    grid = N_SPOT // blk
    bs_h = lambda: pl.BlockSpec((blk, HW), lambda i: (i, 0))
    return pl.pallas_call(
        body,
        grid=(grid,),
        in_specs=[
            bs_h(), bs_h(), bs_h(), bs_h(),
            pl.BlockSpec((HID, HID), lambda i: (0, 0)),
            pl.BlockSpec((1, HID), lambda i: (0, 0)),
            pl.BlockSpec((1, HID), lambda i: (0, 0)),
        ],
        out_specs=pl.BlockSpec((8, 128), lambda i: (0, 0)),
        out_shape=jax.ShapeDtypeStruct((8, 128), jnp.float32),
    )(ss0, ss1, gs0, gs1, kW, kb2, q2)


def _final_linear(ss0, ss1, gs0, gs1, score, lW, lb2, blk):
    def body(s0_ref, s1_ref, g0_ref, g1_ref, sc_ref, lw_ref, lb_ref, o_ref):
        s0 = sc_ref[0, 0] / N_SPOT
        s1 = sc_ref[1, 0] / N_SPOT
        m = jnp.maximum(s0, s1)
        e0 = jnp.exp(s0 - m)
        e1 = jnp.exp(s1 - m)
        w0 = e0 / (e0 + e1)
        w1 = 1.0 - w0
        ss = jnp.concatenate([s0_ref[...], s1_ref[...]], axis=1)
        gs = jnp.concatenate([g0_ref[...], g1_ref[...]], axis=1)
        comb = ss * w0 + gs * w1
        o_ref[...] = jnp.dot(comb, lw_ref[...],
                             preferred_element_type=jnp.float32) + lb_ref[...]

    grid = N_SPOT // blk
    bs_h = lambda: pl.BlockSpec((blk, HW), lambda i: (i, 0))
    return pl.pallas_call(
        body,
        grid=(grid,),
        in_specs=[
            bs_h(), bs_h(), bs_h(), bs_h(),
            pl.BlockSpec((8, 128), lambda i: (0, 0)),
            pl.BlockSpec((HID, OUT_CH), lambda i: (0, 0)),
            pl.BlockSpec((1, OUT_CH), lambda i: (0, 0)),
        ],
        out_specs=pl.BlockSpec((blk, OUT_CH), lambda i: (i, 0)),
        out_shape=jax.ShapeDtypeStruct((N_SPOT, OUT_CH), jnp.float32),
    )(ss0, ss1, gs0, gs1, score, lW, lb2)


def _alpha_proj(att):
    eye = jnp.eye(HEADS, dtype=jnp.float32)
    return (eye[:, None, :] * att[:, :, None]).reshape(HID, HEADS)


def kernel(x_spot, x_gene, edge_index_ss, edge_src_gs, edge_dst_gs,
           proj_spot_W, proj_spot_b, proj_gene_W, proj_gene_b,
           att_src_ss, att_dst_ss, att_src_gs, att_dst_gs,
           k_lin_W, k_lin_b, q, lin_W, lin_b):
    f32 = jnp.float32
    i32 = jnp.int32
    src_ss = edge_index_ss[0].astype(i32)
    dst_ss = edge_index_ss[1].astype(i32)
    src_gs = edge_src_gs.astype(i32)
    dst_gs = edge_dst_gs.astype(i32)

    zcols = jnp.zeros((HID, HEADS), f32)
    A_spot = jnp.concatenate(
        [_alpha_proj(att_src_ss), _alpha_proj(att_dst_ss),
         _alpha_proj(att_dst_gs), zcols], axis=1)
    A_gene = jnp.concatenate(
        [_alpha_proj(att_src_gs), zcols, zcols, zcols], axis=1)

    hs0, hs1, alpha_spot = _prep_nodes(
        x_spot.astype(f32), proj_spot_W, proj_spot_b.reshape(1, HID),
        A_spot, N_SPOT, 400)
    hg0, hg1, alpha_gene = _prep_nodes(
        x_gene.astype(f32), proj_gene_W, proj_gene_b.reshape(1, HID),
        A_gene, N_GENE, 2000)

    def _sd(src, dst):
        pad = E_PAD - E_EDGE
        src_p = jnp.concatenate([src, jnp.zeros((pad,), i32)])
        dst_p = jnp.concatenate([dst, jnp.full((pad,), N_SPOT, i32)])
        return jnp.stack([src_p.reshape(-1, CH), dst_p.reshape(-1, CH)],
                         axis=1).reshape(-1)

    ek_ss = _make_edge_kernel(4)
    ek_gs = _make_edge_kernel(8)
    ss0, ss1 = ek_ss(hs0, hs1, alpha_spot, alpha_spot, _sd(src_ss, dst_ss))
    gs0, gs1 = ek_gs(hg0, hg1, alpha_gene, alpha_spot, _sd(src_gs, dst_gs))

    score = _semantic_score(ss0, ss1, gs0, gs1, k_lin_W,
                            k_lin_b.reshape(1, HID), q.reshape(1, HID), 400)
    return _final_linear(ss0, ss1, gs0, gs1, score, lin_W,
                         lin_b.reshape(1, OUT_CH), 400)


# speedup vs baseline: 1.0038x; 1.0038x over previous
import functools

import jax
import jax.numpy as jnp
from jax import lax
from jax.experimental import pallas as pl
from jax.experimental.pallas import tpu as pltpu
from jax.experimental.pallas import tpu_sc as plsc

N_SPOT, N_GENE = 50000, 2000
E_EDGE = 800000
IN_CH, HID, HEADS, DH = 128, 64, 4, 16
OUT_CH = 16

HALF = N_SPOT // 2
TRASH = HALF
ACC_R = HALF + 8
ROW_W = 48
HSW = 48
HW = 32
CH = 128
CT = 400
E_PAD = CT * 16 * CH
SUP = 16
NSUP = CT // SUP
SDW = SUP * 2 * CH
ROWB = 40
NBLK = HALF // ROWB


def _make_edge_kernel(sel_off):
    mesh = plsc.VectorSubcoreMesh(core_axis_name="c", subcore_axis_name="s")

    @functools.partial(
        pl.kernel,
        out_type=[jax.ShapeDtypeStruct((N_SPOT, HW), jnp.float32),
                  jax.ShapeDtypeStruct((N_SPOT, HW), jnp.float32)],
        mesh=mesh,
        scratch_types=[
            pltpu.VMEM((2 * SDW,), jnp.int32),
            pltpu.VMEM((2, CH), jnp.int32),
            pltpu.VMEM((2 * CH, HSW), jnp.float32),
            pltpu.VMEM((2 * CH, 16), jnp.float32),
            pltpu.VMEM((2 * CH, ROW_W), jnp.float32),
            pltpu.VMEM((ROWB, ROW_W), jnp.float32),
            pltpu.VMEM((ROWB, HW), jnp.float32),
            pltpu.VMEM_SHARED((ACC_R, ROW_W), jnp.float32),
            pltpu.SemaphoreType.DMA,
            pltpu.SemaphoreType.DMA((2,)),
            pltpu.SemaphoreType.DMA((2,)),
            pltpu.SemaphoreType.DMA((2,)),
        ],
        compiler_params=pltpu.CompilerParams(use_tc_tiling_on_sc=False),
    )
    def ek(h01_hbm, h23_hbm, adst_hbm, sd_hbm,
           out0_hbm, out1_hbm,
           sd_v, dstc_v, h_v, ad_v, msg_v, fin_v, fo_v,
           acc, sem_sd, sem_h, sem_d, sem_sc):
        c = lax.axis_index("c")
        sid = lax.axis_index("s")
        lo = c * HALF
        tile_base = sid * CT * 2 * CH

        btrips = 39 + jnp.where(sid < NBLK - 39 * 16, 1, 0)
        sel_a = lax.iota(jnp.int32, 16) % 4 + sel_off
        zv = jnp.zeros((16,), jnp.float32)

        def sd_off(j):
            return ((j // SUP) % 2) * SDW + (j % SUP) * 2 * CH

        def gathers(j, par, h_hbm, wait):
            off = sd_off(j)
            src_idx = sd_v.at[pl.ds(off, CH)]
            dst_idx = sd_v.at[pl.ds(off + CH, CH)]
            for tab, idx, buf, sem in (
                    (h_hbm, src_idx, h_v, sem_h),
                    (adst_hbm, dst_idx, ad_v, sem_d)):
                if wait:
                    pltpu.make_async_copy(tab.at[idx],
                                          buf.at[pl.ds(par * CH, CH)],
                                          sem.at[par]).wait()
                else:
                    pltpu.async_copy(tab.at[idx],
                                     buf.at[pl.ds(par * CH, CH)],
                                     sem.at[par])

        def scatter_wait(par):
            pltpu.make_async_copy(msg_v.at[pl.ds(par * CH, CH)],
                                  acc.at[dstc_v.at[par]],
                                  sem_sc.at[par]).wait()

        for phase, (h_hbm, out_hbm) in enumerate(
                ((h01_hbm, out0_hbm), (h23_hbm, out1_hbm))):
            for r in range(ROWB):
                for g0 in (0, 16, 32):
                    fin_v[r, pl.ds(g0, 16)] = zv

            def zblk(j, carry):
                b = sid + 16 * j
                pltpu.sync_copy(fin_v, acc.at[pl.ds(b * ROWB, ROWB)])
                return carry

            lax.fori_loop(0, btrips, zblk, 0)
            plsc.subcore_barrier()

            pltpu.sync_copy(sd_hbm.at[pl.ds(tile_base, SDW)],
                            sd_v.at[pl.ds(0, SDW)])
            pltpu.async_copy(sd_hbm.at[pl.ds(tile_base + SDW, SDW)],
                             sd_v.at[pl.ds(SDW, SDW)], sem_sd)
            gathers(0, 0, h_hbm, wait=False)

            def chunk_body(j, carry):
                par = j % 2
                npar = 1 - par

                @pl.when(j >= 2)
                def _():
                    scatter_wait(par)

                @pl.when(j < CT - 1)
                def _():
                    @pl.when((j + 1) % SUP == 0)
                    def _():
                        sp = ((j + 1) // SUP) % 2
                        pltpu.make_async_copy(
                            sd_hbm.at[pl.ds(
                                tile_base + ((j + 1) // SUP) * SDW, SDW)],
                            sd_v.at[pl.ds(sp * SDW, SDW)], sem_sd).wait()
                    gathers(j + 1, npar, h_hbm, wait=False)

                gathers(j, par, h_hbm, wait=True)

                doff = sd_off(j) + CH
                for g in range(CH // 16):
                    d = sd_v[pl.ds(doff + g * 16, 16)] - lo
                    ok = (d >= 0) & (d < HALF)
                    dstc_v[par, pl.ds(g * 16, 16)] = jnp.where(ok, d, TRASH)

                def edge_body(e4, carry2):
                    for u in range(4):
                        e = par * CH + e4 * 4 + u
                        ad_al = ad_v[e, :].at[sel_a].get(
                            mode="promise_in_bounds")
                        a = h_v[e, pl.ds(HW, 16)] + ad_al
                        a = jnp.maximum(a, 0.2 * a)
                        ex = jnp.exp(a)
                        msg_v[e, pl.ds(HW, 16)] = ex
                        for hh in range(2):
                            bi = jnp.full((16,), 2 * phase + hh, jnp.int32)
                            w = ex.at[bi].get(mode="promise_in_bounds")
                            msg_v[e, pl.ds(DH * hh, 16)] = (
                                h_v[e, pl.ds(DH * hh, 16)] * w)
                    return carry2

                lax.fori_loop(0, CH // 4, edge_body, 0)
                pltpu.async_copy(msg_v.at[pl.ds(par * CH, CH)],
                                 acc.at[dstc_v.at[par]],
                                 sem_sc.at[par], add=True)

                @pl.when(((j + 1) % SUP == 0) & (j + SUP + 1 < CT))
                def _():
                    s1 = (j + 1) // SUP + 1
                    pltpu.async_copy(
                        sd_hbm.at[pl.ds(tile_base + s1 * SDW, SDW)],
                        sd_v.at[pl.ds((s1 % 2) * SDW, SDW)], sem_sd)

                return carry

            lax.fori_loop(0, CT, chunk_body, 0)
            scatter_wait(0)
            scatter_wait(1)
            plsc.subcore_barrier()

            def fin_body(j, carry):
                b = sid + 16 * j
                r0 = b * ROWB
                pltpu.sync_copy(acc.at[pl.ds(r0, ROWB)], fin_v)

                def row_body(r, carry2):
                    exr = fin_v[r, pl.ds(HW, 16)]
                    for hh in range(2):
                        bi = jnp.full((16,), 2 * phase + hh, jnp.int32)
                        den = exr.at[bi].get(mode="promise_in_bounds") + 1e-16
                        o = fin_v[r, pl.ds(DH * hh, 16)] / den
                        fo_v[r, pl.ds(DH * hh, 16)] = jnp.maximum(o, 0.0)
                    return carry2

                lax.fori_loop(0, ROWB, row_body, 0)
                pltpu.sync_copy(fo_v, out_hbm.at[pl.ds(lo + r0, ROWB)])
                return carry

            lax.fori_loop(0, btrips, fin_body, 0)

    return ek


def _prep_nodes(x, W, b2, A, n_rows, blk):
    def body(x_ref, w_ref, b_ref, a_ref, h0_ref, h1_ref, al_ref):
        h = jnp.dot(x_ref[...], w_ref[...],
                    preferred_element_type=jnp.float32) + b_ref[...]
        al = jnp.dot(h, a_ref[...], preferred_element_type=jnp.float32)
        h0_ref[...] = jnp.concatenate([h[:, :HW], al], axis=1)
        h1_ref[...] = jnp.concatenate([h[:, HW:], al], axis=1)
        al_ref[...] = al

    grid = n_rows // blk
    return pl.pallas_call(
        body,
        grid=(grid,),
        in_specs=[
            pl.BlockSpec((blk, IN_CH), lambda i: (i, 0)),
            pl.BlockSpec((IN_CH, HID), lambda i: (0, 0)),
            pl.BlockSpec((1, HID), lambda i: (0, 0)),
            pl.BlockSpec((HID, 16), lambda i: (0, 0)),
        ],
        out_specs=[
            pl.BlockSpec((blk, HSW), lambda i: (i, 0)),
            pl.BlockSpec((blk, HSW), lambda i: (i, 0)),
            pl.BlockSpec((blk, 16), lambda i: (i, 0)),
        ],
        out_shape=[
            jax.ShapeDtypeStruct((n_rows, HSW), jnp.float32),
            jax.ShapeDtypeStruct((n_rows, HSW), jnp.float32),
            jax.ShapeDtypeStruct((n_rows, 16), jnp.float32),
        ],
    )(x, W, b2, A)


def _semantic_score(ss0, ss1, gs0, gs1, kW, kb2, q2, blk):
    def body(s0_ref, s1_ref, g0_ref, g1_ref, kw_ref, kb_ref, q_ref, o_ref):
        i = pl.program_id(0)

        @pl.when(i == 0)
        def _():
            o_ref[...] = jnp.zeros_like(o_ref)

        ss = jnp.concatenate([s0_ref[...], s1_ref[...]], axis=1)
        gs = jnp.concatenate([g0_ref[...], g1_ref[...]], axis=1)
        t0 = jnp.tanh(jnp.dot(ss, kw_ref[...],
                              preferred_element_type=jnp.float32) + kb_ref[...])
        t1 = jnp.tanh(jnp.dot(gs, kw_ref[...],
                              preferred_element_type=jnp.float32) + kb_ref[...])
        sc0 = jnp.sum(t0 * q_ref[...])
        sc1 = jnp.sum(t1 * q_ref[...])
        rows = lax.broadcasted_iota(jnp.int32, (8, 128), 0)
        upd = jnp.where(rows == 0, sc0, 0.0) + jnp.where(rows == 1, sc1, 0.0)
        o_ref[...] = o_ref[...] + upd

    grid = N_SPOT // blk
    bs_h = lambda: pl.BlockSpec((blk, HW), lambda i: (i, 0))
    return pl.pallas_call(
        body,
        grid=(grid,),
        in_specs=[
            bs_h(), bs_h(), bs_h(), bs_h(),
            pl.BlockSpec((HID, HID), lambda i: (0, 0)),
            pl.BlockSpec((1, HID), lambda i: (0, 0)),
            pl.BlockSpec((1, HID), lambda i: (0, 0)),
        ],
        out_specs=pl.BlockSpec((8, 128), lambda i: (0, 0)),
        out_shape=jax.ShapeDtypeStruct((8, 128), jnp.float32),
    )(ss0, ss1, gs0, gs1, kW, kb2, q2)


def _final_linear(ss0, ss1, gs0, gs1, score, lW, lb2, blk):
    def body(s0_ref, s1_ref, g0_ref, g1_ref, sc_ref, lw_ref, lb_ref, o_ref):
        s0 = sc_ref[0, 0] / N_SPOT
        s1 = sc_ref[1, 0] / N_SPOT
        m = jnp.maximum(s0, s1)
        e0 = jnp.exp(s0 - m)
        e1 = jnp.exp(s1 - m)
        w0 = e0 / (e0 + e1)
        w1 = 1.0 - w0
        ss = jnp.concatenate([s0_ref[...], s1_ref[...]], axis=1)
        gs = jnp.concatenate([g0_ref[...], g1_ref[...]], axis=1)
        comb = ss * w0 + gs * w1
        o_ref[...] = jnp.dot(comb, lw_ref[...],
                             preferred_element_type=jnp.float32) + lb_ref[...]

    grid = N_SPOT // blk
    bs_h = lambda: pl.BlockSpec((blk, HW), lambda i: (i, 0))
    return pl.pallas_call(
        body,
        grid=(grid,),
        in_specs=[
            bs_h(), bs_h(), bs_h(), bs_h(),
            pl.BlockSpec((8, 128), lambda i: (0, 0)),
            pl.BlockSpec((HID, OUT_CH), lambda i: (0, 0)),
            pl.BlockSpec((1, OUT_CH), lambda i: (0, 0)),
        ],
        out_specs=pl.BlockSpec((blk, OUT_CH), lambda i: (i, 0)),
        out_shape=jax.ShapeDtypeStruct((N_SPOT, OUT_CH), jnp.float32),
    )(ss0, ss1, gs0, gs1, score, lW, lb2)


def _alpha_proj(att):
    eye = jnp.eye(HEADS, dtype=jnp.float32)
    return (eye[:, None, :] * att[:, :, None]).reshape(HID, HEADS)


def kernel(x_spot, x_gene, edge_index_ss, edge_src_gs, edge_dst_gs,
           proj_spot_W, proj_spot_b, proj_gene_W, proj_gene_b,
           att_src_ss, att_dst_ss, att_src_gs, att_dst_gs,
           k_lin_W, k_lin_b, q, lin_W, lin_b):
    f32 = jnp.float32
    i32 = jnp.int32
    src_ss = edge_index_ss[0].astype(i32)
    dst_ss = edge_index_ss[1].astype(i32)
    src_gs = edge_src_gs.astype(i32)
    dst_gs = edge_dst_gs.astype(i32)

    zcols = jnp.zeros((HID, HEADS), f32)
    A_spot = jnp.concatenate(
        [_alpha_proj(att_src_ss), _alpha_proj(att_dst_ss),
         _alpha_proj(att_dst_gs), zcols], axis=1)
    A_gene = jnp.concatenate(
        [_alpha_proj(att_src_gs), zcols, zcols, zcols], axis=1)

    hs0, hs1, alpha_spot = _prep_nodes(
        x_spot.astype(f32), proj_spot_W, proj_spot_b.reshape(1, HID),
        A_spot, N_SPOT, 400)
    hg0, hg1, alpha_gene = _prep_nodes(
        x_gene.astype(f32), proj_gene_W, proj_gene_b.reshape(1, HID),
        A_gene, N_GENE, 2000)

    def _sd(src, dst):
        pad = E_PAD - E_EDGE
        src_p = jnp.concatenate([src, jnp.zeros((pad,), i32)])
        dst_p = jnp.concatenate([dst, jnp.full((pad,), N_SPOT, i32)])
        return jnp.stack([src_p.reshape(-1, CH), dst_p.reshape(-1, CH)],
                         axis=1).reshape(-1)

    ek_ss = _make_edge_kernel(4)
    ek_gs = _make_edge_kernel(8)
    ss0, ss1 = ek_ss(hs0, hs1, alpha_spot, _sd(src_ss, dst_ss))
    gs0, gs1 = ek_gs(hg0, hg1, alpha_spot, _sd(src_gs, dst_gs))

    score = _semantic_score(ss0, ss1, gs0, gs1, k_lin_W,
                            k_lin_b.reshape(1, HID), q.reshape(1, HID), 400)
    return _final_linear(ss0, ss1, gs0, gs1, score, lin_W,
                         lin_b.reshape(1, OUT_CH), 400)


# speedup vs baseline: 1.0676x; 1.0636x over previous
import functools

import jax
import jax.numpy as jnp
from jax import lax
from jax.experimental import pallas as pl
from jax.experimental.pallas import tpu as pltpu
from jax.experimental.pallas import tpu_sc as plsc

N_SPOT, N_GENE = 50000, 2000
E_EDGE = 800000
IN_CH, HID, HEADS, DH = 128, 64, 4, 16
OUT_CH = 16

HALF = N_SPOT // 2
TRASH = HALF
ACC_R = HALF + 8
ROW_W = 48
HW = 32
CH = 128
CT = 400
E_PAD = CT * 16 * CH
SUP = 16
NSUP = CT // SUP
SDW = SUP * 2 * CH
ROWB = 40
NBLK = HALF // ROWB


def _make_edge_kernel(sel_off):
    mesh = plsc.VectorSubcoreMesh(core_axis_name="c", subcore_axis_name="s")

    @functools.partial(
        pl.kernel,
        out_type=[jax.ShapeDtypeStruct((N_SPOT, HW), jnp.float32),
                  jax.ShapeDtypeStruct((N_SPOT, HW), jnp.float32)],
        mesh=mesh,
        scratch_types=[
            pltpu.VMEM((2 * SDW,), jnp.int32),
            pltpu.VMEM((2, CH), jnp.int32),
            pltpu.VMEM((2 * CH, HW), jnp.float32),
            pltpu.VMEM((2 * CH, 16), jnp.float32),
            pltpu.VMEM((2 * CH, 16), jnp.float32),
            pltpu.VMEM((2 * CH, ROW_W), jnp.float32),
            pltpu.VMEM((ROWB, ROW_W), jnp.float32),
            pltpu.VMEM((ROWB, HW), jnp.float32),
            pltpu.VMEM_SHARED((ACC_R, ROW_W), jnp.float32),
            pltpu.SemaphoreType.DMA,
            pltpu.SemaphoreType.DMA((2,)),
            pltpu.SemaphoreType.DMA((2,)),
            pltpu.SemaphoreType.DMA((2,)),
            pltpu.SemaphoreType.DMA((2,)),
        ],
        compiler_params=pltpu.CompilerParams(use_tc_tiling_on_sc=False),
    )
    def ek(h01_hbm, h23_hbm, asrc_hbm, adst_hbm, sd_hbm,
           out0_hbm, out1_hbm,
           sd_v, dstc_v, h_v, as_v, ad_v, msg_v, fin_v, fo_v,
           acc, sem_sd, sem_h, sem_a, sem_d, sem_sc):
        c = lax.axis_index("c")
        sid = lax.axis_index("s")
        lo = c * HALF
        tile_base = sid * CT * 2 * CH

        btrips = 39 + jnp.where(sid < NBLK - 39 * 16, 1, 0)
        sel_a = lax.iota(jnp.int32, 16) % 4 + sel_off
        zv = jnp.zeros((16,), jnp.float32)

        def sd_off(j):
            return ((j // SUP) % 2) * SDW + (j % SUP) * 2 * CH

        def gathers(j, par, h_hbm, wait):
            off = sd_off(j)
            src_idx = sd_v.at[pl.ds(off, CH)]
            dst_idx = sd_v.at[pl.ds(off + CH, CH)]
            for tab, idx, buf, sem in (
                    (h_hbm, src_idx, h_v, sem_h),
                    (asrc_hbm, src_idx, as_v, sem_a),
                    (adst_hbm, dst_idx, ad_v, sem_d)):
                if wait:
                    pltpu.make_async_copy(tab.at[idx],
                                          buf.at[pl.ds(par * CH, CH)],
                                          sem.at[par]).wait()
                else:
                    pltpu.async_copy(tab.at[idx],
                                     buf.at[pl.ds(par * CH, CH)],
                                     sem.at[par])

        def scatter_wait(par):
            pltpu.make_async_copy(msg_v.at[pl.ds(par * CH, CH)],
                                  acc.at[dstc_v.at[par]],
                                  sem_sc.at[par]).wait()

        for phase, (h_hbm, out_hbm) in enumerate(
                ((h01_hbm, out0_hbm), (h23_hbm, out1_hbm))):
            for r in range(ROWB):
                for g0 in (0, 16, 32):
                    fin_v[r, pl.ds(g0, 16)] = zv

            def zblk(j, carry):
                b = sid + 16 * j
                pltpu.sync_copy(fin_v, acc.at[pl.ds(b * ROWB, ROWB)])
                return carry

            lax.fori_loop(0, btrips, zblk, 0)
            plsc.subcore_barrier()

            pltpu.sync_copy(sd_hbm.at[pl.ds(tile_base, SDW)],
                            sd_v.at[pl.ds(0, SDW)])
            pltpu.async_copy(sd_hbm.at[pl.ds(tile_base + SDW, SDW)],
                             sd_v.at[pl.ds(SDW, SDW)], sem_sd)
            gathers(0, 0, h_hbm, wait=False)

            def chunk_body(j, carry):
                par = j % 2
                npar = 1 - par

                @pl.when(j >= 2)
                def _():
                    scatter_wait(par)

                @pl.when(j < CT - 1)
                def _():
                    @pl.when((j + 1) % SUP == 0)
                    def _():
                        sp = ((j + 1) // SUP) % 2
                        pltpu.make_async_copy(
                            sd_hbm.at[pl.ds(
                                tile_base + ((j + 1) // SUP) * SDW, SDW)],
                            sd_v.at[pl.ds(sp * SDW, SDW)], sem_sd).wait()
                    gathers(j + 1, npar, h_hbm, wait=False)

                gathers(j, par, h_hbm, wait=True)

                doff = sd_off(j) + CH
                for g in range(CH // 16):
                    d = sd_v[pl.ds(doff + g * 16, 16)] - lo
                    ok = (d >= 0) & (d < HALF)
                    dstc_v[par, pl.ds(g * 16, 16)] = jnp.where(ok, d, TRASH)

                def edge_body(e4, carry2):
                    for u in range(4):
                        e = par * CH + e4 * 4 + u
                        ad_al = ad_v[e, :].at[sel_a].get(
                            mode="promise_in_bounds")
                        a = as_v[e, :] + ad_al
                        a = jnp.maximum(a, 0.2 * a)
                        ex = jnp.exp(a)
                        msg_v[e, pl.ds(HW, 16)] = ex
                        for hh in range(2):
                            bi = jnp.full((16,), 2 * phase + hh, jnp.int32)
                            w = ex.at[bi].get(mode="promise_in_bounds")
                            msg_v[e, pl.ds(DH * hh, 16)] = (
                                h_v[e, pl.ds(DH * hh, 16)] * w)
                    return carry2

                lax.fori_loop(0, CH // 4, edge_body, 0)
                pltpu.async_copy(msg_v.at[pl.ds(par * CH, CH)],
                                 acc.at[dstc_v.at[par]],
                                 sem_sc.at[par], add=True)

                @pl.when(((j + 1) % SUP == 0) & (j + SUP + 1 < CT))
                def _():
                    s1 = (j + 1) // SUP + 1
                    pltpu.async_copy(
                        sd_hbm.at[pl.ds(tile_base + s1 * SDW, SDW)],
                        sd_v.at[pl.ds((s1 % 2) * SDW, SDW)], sem_sd)

                return carry

            lax.fori_loop(0, CT, chunk_body, 0)
            scatter_wait(0)
            scatter_wait(1)
            plsc.subcore_barrier()

            def fin_body(j, carry):
                b = sid + 16 * j
                r0 = b * ROWB
                pltpu.sync_copy(acc.at[pl.ds(r0, ROWB)], fin_v)

                def row_body(r, carry2):
                    exr = fin_v[r, pl.ds(HW, 16)]
                    for hh in range(2):
                        bi = jnp.full((16,), 2 * phase + hh, jnp.int32)
                        den = exr.at[bi].get(mode="promise_in_bounds") + 1e-16
                        o = fin_v[r, pl.ds(DH * hh, 16)] / den
                        fo_v[r, pl.ds(DH * hh, 16)] = jnp.maximum(o, 0.0)
                    return carry2

                lax.fori_loop(0, ROWB, row_body, 0)
                pltpu.sync_copy(fo_v, out_hbm.at[pl.ds(lo + r0, ROWB)])
                return carry

            lax.fori_loop(0, btrips, fin_body, 0)

    return ek


def _prep_nodes(x, W, b2, A, n_rows, blk):
    def body(x_ref, w_ref, b_ref, a_ref, h0_ref, h1_ref, al_ref):
        h = jnp.dot(x_ref[...], w_ref[...],
                    preferred_element_type=jnp.float32) + b_ref[...]
        h0_ref[...] = h[:, :HW]
        h1_ref[...] = h[:, HW:]
        al_ref[...] = jnp.dot(h, a_ref[...], preferred_element_type=jnp.float32)

    grid = n_rows // blk
    return pl.pallas_call(
        body,
        grid=(grid,),
        in_specs=[
            pl.BlockSpec((blk, IN_CH), lambda i: (i, 0)),
            pl.BlockSpec((IN_CH, HID), lambda i: (0, 0)),
            pl.BlockSpec((1, HID), lambda i: (0, 0)),
            pl.BlockSpec((HID, 16), lambda i: (0, 0)),
        ],
        out_specs=[
            pl.BlockSpec((blk, HW), lambda i: (i, 0)),
            pl.BlockSpec((blk, HW), lambda i: (i, 0)),
            pl.BlockSpec((blk, 16), lambda i: (i, 0)),
        ],
        out_shape=[
            jax.ShapeDtypeStruct((n_rows, HW), jnp.float32),
            jax.ShapeDtypeStruct((n_rows, HW), jnp.float32),
            jax.ShapeDtypeStruct((n_rows, 16), jnp.float32),
        ],
    )(x, W, b2, A)


def _semantic_score(ss0, ss1, gs0, gs1, kW, kb2, q2, blk):
    def body(s0_ref, s1_ref, g0_ref, g1_ref, kw_ref, kb_ref, q_ref, o_ref):
        i = pl.program_id(0)

        @pl.when(i == 0)
        def _():
            o_ref[...] = jnp.zeros_like(o_ref)

        ss = jnp.concatenate([s0_ref[...], s1_ref[...]], axis=1)
        gs = jnp.concatenate([g0_ref[...], g1_ref[...]], axis=1)
        t0 = jnp.tanh(jnp.dot(ss, kw_ref[...],
                              preferred_element_type=jnp.float32) + kb_ref[...])
        t1 = jnp.tanh(jnp.dot(gs, kw_ref[...],
                              preferred_element_type=jnp.float32) + kb_ref[...])
        sc0 = jnp.sum(t0 * q_ref[...])
        sc1 = jnp.sum(t1 * q_ref[...])
        rows = lax.broadcasted_iota(jnp.int32, (8, 128), 0)
        upd = jnp.where(rows == 0, sc0, 0.0) + jnp.where(rows == 1, sc1, 0.0)
        o_ref[...] = o_ref[...] + upd

    grid = N_SPOT // blk
    bs_h = lambda: pl.BlockSpec((blk, HW), lambda i: (i, 0))
    return pl.pallas_call(
        body,
        grid=(grid,),
        in_specs=[
            bs_h(), bs_h(), bs_h(), bs_h(),
            pl.BlockSpec((HID, HID), lambda i: (0, 0)),
            pl.BlockSpec((1, HID), lambda i: (0, 0)),
            pl.BlockSpec((1, HID), lambda i: (0, 0)),
        ],
        out_specs=pl.BlockSpec((8, 128), lambda i: (0, 0)),
        out_shape=jax.ShapeDtypeStruct((8, 128), jnp.float32),
    )(ss0, ss1, gs0, gs1, kW, kb2, q2)


def _final_linear(ss0, ss1, gs0, gs1, score, lW, lb2, blk):
    def body(s0_ref, s1_ref, g0_ref, g1_ref, sc_ref, lw_ref, lb_ref, o_ref):
        s0 = sc_ref[0, 0] / N_SPOT
        s1 = sc_ref[1, 0] / N_SPOT
        m = jnp.maximum(s0, s1)
        e0 = jnp.exp(s0 - m)
        e1 = jnp.exp(s1 - m)
        w0 = e0 / (e0 + e1)
        w1 = 1.0 - w0
        ss = jnp.concatenate([s0_ref[...], s1_ref[...]], axis=1)
        gs = jnp.concatenate([g0_ref[...], g1_ref[...]], axis=1)
        comb = ss * w0 + gs * w1
        o_ref[...] = jnp.dot(comb, lw_ref[...],
                             preferred_element_type=jnp.float32) + lb_ref[...]

    grid = N_SPOT // blk
    bs_h = lambda: pl.BlockSpec((blk, HW), lambda i: (i, 0))
    return pl.pallas_call(
        body,
        grid=(grid,),
        in_specs=[
            bs_h(), bs_h(), bs_h(), bs_h(),
            pl.BlockSpec((8, 128), lambda i: (0, 0)),
            pl.BlockSpec((HID, OUT_CH), lambda i: (0, 0)),
            pl.BlockSpec((1, OUT_CH), lambda i: (0, 0)),
        ],
        out_specs=pl.BlockSpec((blk, OUT_CH), lambda i: (i, 0)),
        out_shape=jax.ShapeDtypeStruct((N_SPOT, OUT_CH), jnp.float32),
    )(ss0, ss1, gs0, gs1, score, lW, lb2)


def _alpha_proj(att):
    eye = jnp.eye(HEADS, dtype=jnp.float32)
    return (eye[:, None, :] * att[:, :, None]).reshape(HID, HEADS)


def kernel(x_spot, x_gene, edge_index_ss, edge_src_gs, edge_dst_gs,
           proj_spot_W, proj_spot_b, proj_gene_W, proj_gene_b,
           att_src_ss, att_dst_ss, att_src_gs, att_dst_gs,
           k_lin_W, k_lin_b, q, lin_W, lin_b):
    f32 = jnp.float32
    i32 = jnp.int32
    src_ss = edge_index_ss[0].astype(i32)
    dst_ss = edge_index_ss[1].astype(i32)
    src_gs = edge_src_gs.astype(i32)
    dst_gs = edge_dst_gs.astype(i32)

    zcols = jnp.zeros((HID, HEADS), f32)
    A_spot = jnp.concatenate(
        [_alpha_proj(att_src_ss), _alpha_proj(att_dst_ss),
         _alpha_proj(att_dst_gs), zcols], axis=1)
    A_gene = jnp.concatenate(
        [_alpha_proj(att_src_gs), zcols, zcols, zcols], axis=1)

    hs0, hs1, alpha_spot = _prep_nodes(
        x_spot.astype(f32), proj_spot_W, proj_spot_b.reshape(1, HID),
        A_spot, N_SPOT, 400)
    hg0, hg1, alpha_gene = _prep_nodes(
        x_gene.astype(f32), proj_gene_W, proj_gene_b.reshape(1, HID),
        A_gene, N_GENE, 2000)

    def _sd(src, dst):
        pad = E_PAD - E_EDGE
        src_p = jnp.concatenate([src, jnp.zeros((pad,), i32)])
        dst_p = jnp.concatenate([dst, jnp.full((pad,), N_SPOT, i32)])
        return jnp.stack([src_p.reshape(-1, CH), dst_p.reshape(-1, CH)],
                         axis=1).reshape(-1)

    ek_ss = _make_edge_kernel(4)
    ek_gs = _make_edge_kernel(8)
    ss0, ss1 = ek_ss(hs0, hs1, alpha_spot, alpha_spot, _sd(src_ss, dst_ss))
    gs0, gs1 = ek_gs(hg0, hg1, alpha_gene, alpha_spot, _sd(src_gs, dst_gs))

    score = _semantic_score(ss0, ss1, gs0, gs1, k_lin_W,
                            k_lin_b.reshape(1, HID), q.reshape(1, HID), 400)
    return _final_linear(ss0, ss1, gs0, gs1, score, lin_W,
                         lin_b.reshape(1, OUT_CH), 400)
